# baseline (speedup 1.0000x reference)
import jax
import jax.numpy as jnp
from jax.experimental import pallas as pl
from jax.experimental.pallas import tpu as pltpu

_B, _C, _H, _W = 32, 384, 64, 64
_HW = _H * _W
_R = 128
_K = 6
_ROWS = _B * _C
_NS = _ROWS // (_R * _K)
_CPR = _C // _R


def _body(stage_ref, x_hbm, e_ref, o_hbm, inbuf, outbuf, insem, outsem):
    del stage_ref
    s = pl.program_id(0)
    base = s * (_R * _K)

    def in_copy(j):
        return pltpu.make_async_copy(
            x_hbm.at[pl.ds(base + j * _R, _R), :],
            inbuf.at[j],
            insem.at[j],
        )

    def out_copy(j):
        return pltpu.make_async_copy(
            outbuf.at[j],
            o_hbm.at[pl.ds(base + j * _R, _R), :],
            outsem.at[j],
        )

    for j in range(_K):
        in_copy(j).start()

    @pl.when(s > 0)
    def _():
        for j in range(_K):
            pltpu.make_async_copy(
                outbuf.at[j],
                o_hbm.at[pl.ds(j * _R, _R), :],
                outsem.at[j],
            ).wait()

    for j in range(_K):
        in_copy(j).wait()
        coff = (j % _CPR) * _R
        outbuf[j] = inbuf[j] * e_ref[0, pl.ds(coff, _R), :]
        out_copy(j).start()

    @pl.when(s == _NS - 1)
    def _():
        for j in range(_K):
            pltpu.make_async_copy(
                outbuf.at[j],
                o_hbm.at[pl.ds(j * _R, _R), :],
                outsem.at[j],
            ).wait()


def kernel(x, stage, embeddings):
    s = jnp.asarray(stage, dtype=jnp.int32).reshape((1,))
    x2 = x.reshape(_ROWS, _HW)
    e3 = embeddings.reshape(embeddings.shape[0], _C, 1)

    grid_spec = pltpu.PrefetchScalarGridSpec(
        num_scalar_prefetch=1,
        grid=(_NS,),
        in_specs=[
            pl.BlockSpec(memory_space=pltpu.MemorySpace.HBM),
            pl.BlockSpec((1, _C, 1), lambda i, st: (st[0], 0, 0)),
        ],
        out_specs=pl.BlockSpec(memory_space=pltpu.MemorySpace.HBM),
        scratch_shapes=[
            pltpu.VMEM((_K, _R, _HW), jnp.float32),
            pltpu.VMEM((_K, _R, _HW), jnp.float32),
            pltpu.SemaphoreType.DMA((_K,)),
            pltpu.SemaphoreType.DMA((_K,)),
        ],
    )

    out = pl.pallas_call(
        _body,
        grid_spec=grid_spec,
        out_shape=jax.ShapeDtypeStruct((_ROWS, _HW), jnp.float32),
        compiler_params=pltpu.CompilerParams(
            dimension_semantics=("arbitrary",),
        ),
    )(s, x2, e3)
    return out.reshape(_B, _C, _H, _W)


# speedup vs baseline: 1.0122x; 1.0122x over previous
import jax
import jax.numpy as jnp
from jax.experimental import pallas as pl
from jax.experimental.pallas import tpu as pltpu

_B, _C, _H, _W = 32, 384, 64, 64
_HW = _H * _W
_G = 128
_N = (_B * _C) // _G
_CPR = _C // _G


def _body(s_ref, e_ref, x_ref, o_ref):
    st = s_ref[0]
    c0 = (pl.program_id(0) % _CPR) * _G
    for g in range(_G):
        o_ref[g] = x_ref[g] * e_ref[st, c0 + g]


def kernel(x, stage, embeddings):
    s = jnp.asarray(stage, dtype=jnp.int32).reshape((1,))
    x4 = x.reshape(_B * _C, _HW // 128, 128)

    grid_spec = pltpu.PrefetchScalarGridSpec(
        num_scalar_prefetch=2,
        grid=(_N,),
        in_specs=[
            pl.BlockSpec((_G, _HW // 128, 128), lambda i, st_r, e_r: (i, 0, 0)),
        ],
        out_specs=pl.BlockSpec((_G, _HW // 128, 128), lambda i, st_r, e_r: (i, 0, 0)),
    )

    out = pl.pallas_call(
        _body,
        grid_spec=grid_spec,
        out_shape=jax.ShapeDtypeStruct((_B * _C, _HW // 128, 128), jnp.float32),
        compiler_params=pltpu.CompilerParams(
            dimension_semantics=("arbitrary",),
        ),
    )(s, embeddings, x4)
    return out.reshape(_B, _C, _H, _W)
